# Initial kernel scaffold; baseline (speedup 1.0000x reference)
import jax
import jax.numpy as jnp
from jax.experimental import pallas as pl


def kernel(x, ln_w, ln_b, W1, W2):
    raise NotImplementedError("write your pallas kernel here")


# speedup vs baseline: 1.7324x; 1.7324x over previous
import functools

import jax
import jax.numpy as jnp
from jax.experimental import pallas as pl
from jax.experimental.pallas import tpu as pltpu

_EMBED = 2048
_HIDDEN = 256
_NBLK = 64
_TOPK = 8
_TEMP = 2.0


def _router_kernel(x_ref, lnw_ref, lnb_ref, w1_ref, w2_ref,
                   probs_ref, idx_ref, w_ref, aux_ref, acc_ref,
                   *, n_tokens):
    i = pl.program_id(0)
    nsteps = pl.num_programs(0)

    @pl.when(i == 0)
    def _init():
        acc_ref[...] = jnp.zeros_like(acc_ref)

    x = x_ref[...]
    mean = jnp.mean(x, axis=-1, keepdims=True)
    xc = x - mean
    var = jnp.mean(xc * xc, axis=-1, keepdims=True)
    xn = xc * jax.lax.rsqrt(var + 1e-5) * lnw_ref[...] + lnb_ref[...]

    h = jnp.dot(xn, w1_ref[...], preferred_element_type=jnp.float32)
    h = 0.5 * h * (1.0 + jax.lax.erf(h * 0.7071067811865476))
    t_inv = 1.0 / max(_TEMP, 0.1)
    logits = jnp.dot(h, w2_ref[...], preferred_element_type=jnp.float32) * t_inv

    logits = logits - jnp.max(logits, axis=-1, keepdims=True)
    e = jnp.exp(logits)
    probs = e / jnp.sum(e, axis=-1, keepdims=True)
    probs_ref[...] = probs

    blk = probs.shape[0]
    iota = jax.lax.broadcasted_iota(jnp.int32, (blk, _NBLK), 1)
    cur = probs
    onehot_sum = jnp.zeros((blk, _NBLK), jnp.float32)
    tops, idxs = [], []
    for _ in range(_TOPK):
        m = jnp.max(cur, axis=-1, keepdims=True)
        idx = jnp.min(jnp.where(cur == m, iota, _NBLK), axis=-1, keepdims=True)
        hit = iota == idx
        onehot_sum = onehot_sum + hit.astype(jnp.float32)
        cur = jnp.where(hit, -1.0, cur)
        tops.append(m)
        idxs.append(idx)
    top_p = jnp.concatenate(tops, axis=-1)
    idx_ref[...] = jnp.concatenate(idxs, axis=-1)
    w_ref[...] = top_p / (jnp.sum(top_p, axis=-1, keepdims=True) + 1e-8)

    acc = acc_ref[...]
    acc_ref[...] = acc + jnp.concatenate(
        [jnp.sum(probs, axis=0, keepdims=True),
         jnp.sum(onehot_sum, axis=0, keepdims=True)], axis=0)

    @pl.when(i == nsteps - 1)
    def _finalize():
        total = acc_ref[...]
        f = total[1:2, :] / (n_tokens * _TOPK + 1e-8)
        p_mean = total[0:1, :] / n_tokens
        aux_ref[...] = jnp.sum(_NBLK * f * p_mean, keepdims=True).reshape(1, 1)


def kernel(x, ln_w, ln_b, W1, W2):
    b, s, d = x.shape
    n = b * s
    blk = 512
    xf = x.reshape(n, d)
    kfn = functools.partial(_router_kernel, n_tokens=n)
    probs, idx, w, aux = pl.pallas_call(
        kfn,
        grid=(n // blk,),
        in_specs=[
            pl.BlockSpec((blk, d), lambda i: (i, 0)),
            pl.BlockSpec((1, d), lambda i: (0, 0)),
            pl.BlockSpec((1, d), lambda i: (0, 0)),
            pl.BlockSpec((d, _HIDDEN), lambda i: (0, 0)),
            pl.BlockSpec((_HIDDEN, _NBLK), lambda i: (0, 0)),
        ],
        out_specs=(
            pl.BlockSpec((blk, _NBLK), lambda i: (i, 0)),
            pl.BlockSpec((blk, _TOPK), lambda i: (i, 0)),
            pl.BlockSpec((blk, _TOPK), lambda i: (i, 0)),
            pl.BlockSpec((1, 1), lambda i: (0, 0)),
        ),
        out_shape=(
            jax.ShapeDtypeStruct((n, _NBLK), jnp.float32),
            jax.ShapeDtypeStruct((n, _TOPK), jnp.int32),
            jax.ShapeDtypeStruct((n, _TOPK), jnp.float32),
            jax.ShapeDtypeStruct((1, 1), jnp.float32),
        ),
        scratch_shapes=[pltpu.VMEM((2, _NBLK), jnp.float32)],
    )(xf, ln_w.reshape(1, d), ln_b.reshape(1, d), W1, W2)
    return (probs.reshape(b, s, _NBLK), idx.reshape(b, s, _TOPK),
            aux[0, 0], w.reshape(b, s, _TOPK))


# speedup vs baseline: 2.1551x; 1.2440x over previous
import functools

import jax
import jax.numpy as jnp
from jax.experimental import pallas as pl
from jax.experimental.pallas import tpu as pltpu

_EMBED = 2048
_HIDDEN = 256
_NBLK = 64
_TOPK = 8
_TEMP = 2.0
_INT_MIN = -(2**31)


def _router_kernel(x_ref, w1p_ref, cb_ref, w2_ref,
                   probs_ref, idx_ref, w_ref, aux_ref, acc_ref,
                   *, n_tokens):
    i = pl.program_id(0)
    nsteps = pl.num_programs(0)

    @pl.when(i == 0)
    def _init():
        acc_ref[...] = jnp.zeros_like(acc_ref)

    x = x_ref[...]
    d = x.shape[-1]
    s1 = jnp.sum(x, axis=-1, keepdims=True)
    s2 = jnp.sum(x * x, axis=-1, keepdims=True)
    mean = s1 * (1.0 / d)
    var = s2 * (1.0 / d) - mean * mean
    a = jax.lax.rsqrt(var + 1e-5)

    g = jnp.dot(x, w1p_ref[...], preferred_element_type=jnp.float32)
    h = g * a + cb_ref[0:1, :] * (-(a * mean)) + cb_ref[1:2, :]
    h = 0.5 * h * (1.0 + jax.lax.erf(h * 0.7071067811865476))

    t_inv = 1.0 / max(_TEMP, 0.1)
    logits = jnp.dot(h, w2_ref[...], preferred_element_type=jnp.float32) * t_inv
    logits = logits - jnp.max(logits, axis=-1, keepdims=True)
    e = jnp.exp(logits)
    probs = e / jnp.sum(e, axis=-1, keepdims=True)
    probs_ref[...] = probs

    blk = probs.shape[0]
    iota = jax.lax.broadcasted_iota(jnp.int32, (blk, _NBLK), 1)
    pb = jax.lax.bitcast_convert_type(probs, jnp.int32)
    cur = jnp.bitwise_or(jnp.bitwise_and(pb, jnp.int32(-64)),
                         jnp.int32(_NBLK - 1) - iota)
    tops, idxs = [], []
    for _ in range(_TOPK):
        m = jnp.max(cur, axis=-1, keepdims=True)
        idxs.append(jnp.int32(_NBLK - 1) - jnp.bitwise_and(m, jnp.int32(63)))
        tops.append(jax.lax.bitcast_convert_type(
            jnp.bitwise_and(m, jnp.int32(-64)), jnp.float32))
        cur = jnp.where(cur == m, jnp.int32(_INT_MIN), cur)
    top_p = jnp.concatenate(tops, axis=-1)
    idx_ref[...] = jnp.concatenate(idxs, axis=-1)
    w_ref[...] = top_p / (jnp.sum(top_p, axis=-1, keepdims=True) + 1e-8)

    sel = (cur < 0).astype(jnp.float32)
    acc = acc_ref[...]
    acc_ref[...] = acc + jnp.concatenate(
        [jnp.sum(probs, axis=0, keepdims=True),
         jnp.sum(sel, axis=0, keepdims=True)], axis=0)

    @pl.when(i == nsteps - 1)
    def _finalize():
        total = acc_ref[...]
        f = total[1:2, :] / (n_tokens * _TOPK + 1e-8)
        p_mean = total[0:1, :] / n_tokens
        aux_ref[...] = jnp.sum(_NBLK * f * p_mean, keepdims=True).reshape(1, 1)


def kernel(x, ln_w, ln_b, W1, W2):
    b, s, d = x.shape
    n = b * s
    blk = 512
    xf = x.reshape(n, d)
    w1p = W1 * ln_w[:, None]
    cb = jnp.concatenate([jnp.sum(w1p, axis=0, keepdims=True),
                          (ln_b @ W1).reshape(1, -1)], axis=0)
    kfn = functools.partial(_router_kernel, n_tokens=n)
    probs, idx, w, aux = pl.pallas_call(
        kfn,
        grid=(n // blk,),
        in_specs=[
            pl.BlockSpec((blk, d), lambda i: (i, 0)),
            pl.BlockSpec((d, _HIDDEN), lambda i: (0, 0)),
            pl.BlockSpec((2, _HIDDEN), lambda i: (0, 0)),
            pl.BlockSpec((_HIDDEN, _NBLK), lambda i: (0, 0)),
        ],
        out_specs=(
            pl.BlockSpec((blk, _NBLK), lambda i: (i, 0)),
            pl.BlockSpec((blk, _TOPK), lambda i: (i, 0)),
            pl.BlockSpec((blk, _TOPK), lambda i: (i, 0)),
            pl.BlockSpec((1, 1), lambda i: (0, 0)),
        ),
        out_shape=(
            jax.ShapeDtypeStruct((n, _NBLK), jnp.float32),
            jax.ShapeDtypeStruct((n, _TOPK), jnp.int32),
            jax.ShapeDtypeStruct((n, _TOPK), jnp.float32),
            jax.ShapeDtypeStruct((1, 1), jnp.float32),
        ),
        scratch_shapes=[pltpu.VMEM((2, _NBLK), jnp.float32)],
    )(xf, w1p, cb, W2)
    return (probs.reshape(b, s, _NBLK), idx.reshape(b, s, _TOPK),
            aux[0, 0], w.reshape(b, s, _TOPK))
